# Initial kernel scaffold; baseline (speedup 1.0000x reference)
import functools

import jax
import jax.numpy as jnp
from jax import lax
from jax.experimental import pallas as pl
from jax.experimental.pallas import tpu as pltpu
from jax.experimental.pallas import tpu_sc as plsc

_f32 = jnp.float32

_N = 10000
_E = 320000
_D = 128
_G = 64
_NP = 10240
_BN = 512
_NBLK = _NP // _BN
_NW = 32
_CH = 128
_EW = _E // _NW
_ECH = -(-_EW // _CH)
_EWP = _ECH * _CH
_RPT = _NP // 16


def _sc_scatter_add(h, src3, dst3):
    mesh = plsc.VectorSubcoreMesh(core_axis_name="c", subcore_axis_name="s")

    @functools.partial(
        pl.kernel,
        mesh=mesh,
        out_type=jax.ShapeDtypeStruct((2, _NP, _D), _f32),
        scratch_types=[
            pltpu.VMEM((_ECH, _CH), jnp.int32),
            pltpu.VMEM((_ECH, _CH), jnp.int32),
            pltpu.VMEM((_CH, _D), _f32),
            pltpu.VMEM((16, _D), _f32),
            pltpu.VMEM_SHARED((_NP, _D), _f32),
            pltpu.SemaphoreType.DMA,
        ],
    )
    def k(h_hbm, src_hbm, dst_hbm, out_hbm, srcv, dstv, rowsv, zbuf, aggsh, sem):
        cid = lax.axis_index("c")
        sid = lax.axis_index("s")
        wid = sid * 2 + cid
        pltpu.sync_copy(src_hbm.at[wid], srcv)
        pltpu.sync_copy(dst_hbm.at[wid], dstv)
        for r in range(16):
            for l in range(8):
                zbuf[r, pl.ds(l * 16, 16)] = jnp.zeros((16,), _f32)
        base = sid * _RPT

        def zbody(kk, c):
            pltpu.sync_copy(zbuf, aggsh.at[pl.ds(base + kk * 16, 16)])
            return c

        lax.fori_loop(0, _RPT // 16, zbody, 0)
        plsc.subcore_barrier()

        def ebody(j, c):
            pltpu.async_copy(h_hbm.at[srcv.at[j]], rowsv, sem).wait()
            pltpu.sync_copy(rowsv, aggsh.at[dstv.at[j]], add=True)
            return c

        lax.fori_loop(0, _ECH, ebody, 0)
        plsc.subcore_barrier()
        pltpu.sync_copy(aggsh.at[pl.ds(base, _RPT)],
                        out_hbm.at[cid, pl.ds(base, _RPT)])

    return k(h, src3, dst3)


def _row_mask(i):
    rows = i * _BN + lax.broadcasted_iota(jnp.int32, (_BN, 1), 0)
    return (rows < _N).astype(_f32)


def _affine_from_stats(st_ref, g_ref, b_ref):
    mu = st_ref[0:1, :] * (1.0 / _N)
    var = st_ref[1:2, :] * (1.0 / _N) - mu * mu
    cc = g_ref[...] * lax.rsqrt(var + 1e-5)
    dd = b_ref[...] - mu * cc
    return cc, dd


def _c1(h, a0, a1, ep, w1, b1):
    def body(ep_ref, h_ref, a0_ref, a1_ref, w1_ref, b1_ref, a_ref, st_ref, acc_ref):
        i = pl.program_id(0)
        t = ep_ref[0, 0] * h_ref[...] + a0_ref[...] + a1_ref[...]
        a = jnp.dot(t, w1_ref[...], preferred_element_type=_f32) + b1_ref[...]
        a_ref[...] = a
        am = a * _row_mask(i)

        @pl.when(i == 0)
        def _():
            acc_ref[...] = jnp.zeros((2, _D), _f32)

        acc_ref[...] += jnp.concatenate(
            [jnp.sum(am, 0, keepdims=True), jnp.sum(am * am, 0, keepdims=True)], 0)

        @pl.when(i == _NBLK - 1)
        def _():
            st_ref[...] = acc_ref[...]

    return pl.pallas_call(
        body,
        grid=(_NBLK,),
        in_specs=[
            pl.BlockSpec((1, 1), lambda i: (0, 0)),
            pl.BlockSpec((_BN, _D), lambda i: (i, 0)),
            pl.BlockSpec((_BN, _D), lambda i: (i, 0)),
            pl.BlockSpec((_BN, _D), lambda i: (i, 0)),
            pl.BlockSpec((_D, _D), lambda i: (0, 0)),
            pl.BlockSpec((1, _D), lambda i: (0, 0)),
        ],
        out_specs=[
            pl.BlockSpec((_BN, _D), lambda i: (i, 0)),
            pl.BlockSpec((2, _D), lambda i: (0, 0)),
        ],
        out_shape=[
            jax.ShapeDtypeStruct((_NP, _D), _f32),
            jax.ShapeDtypeStruct((2, _D), _f32),
        ],
        scratch_shapes=[pltpu.VMEM((2, _D), _f32)],
        compiler_params=pltpu.CompilerParams(
            dimension_semantics=("arbitrary",)),
    )(ep, h, a0, a1, w1, b1)


def _c2(a, st1, g1, be1, w2, b2):
    def body(a_ref, st_ref, g_ref, be_ref, w2_ref, b2_ref, s_ref, st2_ref, acc_ref):
        i = pl.program_id(0)
        cc, dd = _affine_from_stats(st_ref, g_ref, be_ref)
        r = jnp.maximum(a_ref[...] * cc + dd, 0.0)
        s = jnp.maximum(
            jnp.dot(r, w2_ref[...], preferred_element_type=_f32) + b2_ref[...], 0.0)
        s_ref[...] = s
        sm = s * _row_mask(i)

        @pl.when(i == 0)
        def _():
            acc_ref[...] = jnp.zeros((2, _D), _f32)

        acc_ref[...] += jnp.concatenate(
            [jnp.sum(sm, 0, keepdims=True), jnp.sum(sm * sm, 0, keepdims=True)], 0)

        @pl.when(i == _NBLK - 1)
        def _():
            st2_ref[...] = acc_ref[...]

    return pl.pallas_call(
        body,
        grid=(_NBLK,),
        in_specs=[
            pl.BlockSpec((_BN, _D), lambda i: (i, 0)),
            pl.BlockSpec((2, _D), lambda i: (0, 0)),
            pl.BlockSpec((1, _D), lambda i: (0, 0)),
            pl.BlockSpec((1, _D), lambda i: (0, 0)),
            pl.BlockSpec((_D, _D), lambda i: (0, 0)),
            pl.BlockSpec((1, _D), lambda i: (0, 0)),
        ],
        out_specs=[
            pl.BlockSpec((_BN, _D), lambda i: (i, 0)),
            pl.BlockSpec((2, _D), lambda i: (0, 0)),
        ],
        out_shape=[
            jax.ShapeDtypeStruct((_NP, _D), _f32),
            jax.ShapeDtypeStruct((2, _D), _f32),
        ],
        scratch_shapes=[pltpu.VMEM((2, _D), _f32)],
        compiler_params=pltpu.CompilerParams(
            dimension_semantics=("arbitrary",)),
    )(a, st1, g1, be1, w2, b2)


def _dnorm(s, st2, go, bo):
    def body(s_ref, st_ref, g_ref, b_ref, h_ref):
        cc, dd = _affine_from_stats(st_ref, g_ref, b_ref)
        h_ref[...] = s_ref[...] * cc + dd

    return pl.pallas_call(
        body,
        grid=(_NBLK,),
        in_specs=[
            pl.BlockSpec((_BN, _D), lambda i: (i, 0)),
            pl.BlockSpec((2, _D), lambda i: (0, 0)),
            pl.BlockSpec((1, _D), lambda i: (0, 0)),
            pl.BlockSpec((1, _D), lambda i: (0, 0)),
        ],
        out_specs=pl.BlockSpec((_BN, _D), lambda i: (i, 0)),
        out_shape=jax.ShapeDtypeStruct((_NP, _D), _f32),
        compiler_params=pltpu.CompilerParams(
            dimension_semantics=("arbitrary",)),
    )(s, st2, go, bo)


def _pool(s, st2, go, bo, batch3, wh1, bh1, wh2, bh2):
    def body(s_ref, st_ref, g_ref, b_ref, bt_ref, wh1_ref, bh1_ref, wh2_ref,
             bh2_ref, out_ref, ps_ref, cnt_ref):
        i = pl.program_id(0)
        bb = bt_ref[0, 0, :]
        seg = lax.broadcasted_iota(jnp.int32, (1, _G), 1)
        oh = (bb[:, None] == seg).astype(_f32)

        @pl.when(i == 0)
        def _():
            ps_ref[...] = jnp.zeros((_G, _D), _f32)
            cnt_ref[...] = jnp.zeros((_G, 1), _f32)

        ps_ref[...] += lax.dot_general(
            oh, s_ref[...], (((0,), (0,)), ((), ())), preferred_element_type=_f32)
        cnt_ref[...] += lax.dot_general(
            oh, jnp.ones((_BN, 1), _f32), (((0,), (0,)), ((), ())),
            preferred_element_type=_f32)

        @pl.when(i == _NBLK - 1)
        def _():
            cc, dd = _affine_from_stats(st_ref, g_ref, b_ref)
            pooled = ps_ref[...] * cc + cnt_ref[...] * dd
            hid = jnp.maximum(
                jnp.dot(pooled, wh1_ref[...], preferred_element_type=_f32)
                + bh1_ref[...], 0.0)
            out_ref[...] = (jnp.dot(hid, wh2_ref[...], preferred_element_type=_f32)
                            + bh2_ref[...])

    return pl.pallas_call(
        body,
        grid=(_NBLK,),
        in_specs=[
            pl.BlockSpec((_BN, _D), lambda i: (i, 0)),
            pl.BlockSpec((2, _D), lambda i: (0, 0)),
            pl.BlockSpec((1, _D), lambda i: (0, 0)),
            pl.BlockSpec((1, _D), lambda i: (0, 0)),
            pl.BlockSpec((1, 1, _BN), lambda i: (i, 0, 0)),
            pl.BlockSpec((_D, _G), lambda i: (0, 0)),
            pl.BlockSpec((1, _G), lambda i: (0, 0)),
            pl.BlockSpec((_G, 1), lambda i: (0, 0)),
            pl.BlockSpec((1, 1), lambda i: (0, 0)),
        ],
        out_specs=pl.BlockSpec((_G, 1), lambda i: (0, 0)),
        out_shape=jax.ShapeDtypeStruct((_G, 1), _f32),
        scratch_shapes=[pltpu.VMEM((_G, _D), _f32), pltpu.VMEM((_G, 1), _f32)],
        compiler_params=pltpu.CompilerParams(
            dimension_semantics=("arbitrary",)),
    )(s, st2, go, bo, batch3, wh1, bh1, wh2, bh2)


def kernel(x, edge_index, batch,
           eps_0, W1_0, b1_0, g1_0, be1_0, W2_0, b2_0, go_0, bo_0,
           eps_1, W1_1, b1_1, g1_1, be1_1, W2_1, b2_1, go_1, bo_1,
           eps_2, W1_2, b1_2, g1_2, be1_2, W2_2, b2_2, go_2, bo_2,
           Wh1, bh1, Wh2, bh2):
    src = edge_index[0].reshape(_NW, _EW)
    dst = edge_index[1].reshape(_NW, _EW)
    src3 = jnp.pad(src, ((0, 0), (0, _EWP - _EW))).reshape(_NW, _ECH, _CH)
    dst3 = jnp.pad(dst, ((0, 0), (0, _EWP - _EW)),
                   constant_values=_N).reshape(_NW, _ECH, _CH)
    xp = jnp.pad(x, ((0, _NP - _N), (0, 0)))
    batch3 = jnp.pad(batch, (0, _NP - _N),
                     constant_values=_G).reshape(_NBLK, 1, _BN)

    layers = [
        (eps_0, W1_0, b1_0, g1_0, be1_0, W2_0, b2_0, go_0, bo_0),
        (eps_1, W1_1, b1_1, g1_1, be1_1, W2_1, b2_1, go_1, bo_1),
        (eps_2, W1_2, b1_2, g1_2, be1_2, W2_2, b2_2, go_2, bo_2),
    ]
    h = xp
    out = None
    for i, (eps_i, w1, b1, g1, be1, w2, b2, go, bo) in enumerate(layers):
        agg = _sc_scatter_add(h, src3, dst3)
        ep = (1.0 + eps_i).reshape(1, 1)
        a, st1 = _c1(h, agg[0], agg[1], ep, w1, b1.reshape(1, _D))
        s, st2 = _c2(a, st1, g1.reshape(1, _D), be1.reshape(1, _D),
                     w2, b2.reshape(1, _D))
        if i < 2:
            h = _dnorm(s, st2, go.reshape(1, _D), bo.reshape(1, _D))
        else:
            out = _pool(s, st2, go.reshape(1, _D), bo.reshape(1, _D), batch3,
                        Wh1, bh1.reshape(1, _G), Wh2, bh2.reshape(1, 1))
    return jnp.squeeze(out, -1)


# speedup vs baseline: 4.1606x; 4.1606x over previous
import functools

import jax
import jax.numpy as jnp
from jax import lax
from jax.experimental import pallas as pl
from jax.experimental.pallas import tpu as pltpu
from jax.experimental.pallas import tpu_sc as plsc

_f32 = jnp.float32

_N = 10000
_E = 320000
_D = 128
_G = 64
_NP = 10240
_BN = 512
_NBLK = _NP // _BN
_NW = 32
_CH = 128
_EW = _E // _NW
_ECH = -(-_EW // _CH)
_EWP = _ECH * _CH
_RPT = _NP // 16


def _sc_scatter_add(h, src3, dst3):
    mesh = plsc.VectorSubcoreMesh(core_axis_name="c", subcore_axis_name="s")

    @functools.partial(
        pl.kernel,
        mesh=mesh,
        out_type=jax.ShapeDtypeStruct((2, _NP, _D), _f32),
        scratch_types=[
            pltpu.VMEM((_ECH, _CH), jnp.int32),
            pltpu.VMEM((_ECH, _CH), jnp.int32),
            pltpu.VMEM((_CH, _D), _f32),
            pltpu.VMEM((16, _D), _f32),
            pltpu.VMEM_SHARED((_NP, _D), _f32),
            pltpu.SemaphoreType.DMA,
        ],
    )
    def k(h_hbm, src_hbm, dst_hbm, out_hbm, srcv, dstv, rowsv, zbuf, aggsh, sem):
        cid = lax.axis_index("c")
        sid = lax.axis_index("s")
        wid = sid * 2 + cid
        pltpu.sync_copy(src_hbm.at[wid], srcv)
        pltpu.sync_copy(dst_hbm.at[wid], dstv)
        for r in range(16):
            for l in range(8):
                zbuf[r, pl.ds(l * 16, 16)] = jnp.zeros((16,), _f32)
        base = sid * _RPT

        def zbody(kk, c):
            pltpu.sync_copy(zbuf, aggsh.at[pl.ds(base + kk * 16, 16)])
            return c

        lax.fori_loop(0, _RPT // 16, zbody, 0)
        plsc.subcore_barrier()

        def ebody(j, c):
            pltpu.async_copy(h_hbm.at[srcv.at[j]], rowsv, sem).wait()
            pltpu.sync_copy(rowsv, aggsh.at[dstv.at[j]], add=True)
            return c

        lax.fori_loop(0, _ECH, ebody, 0)
        plsc.subcore_barrier()
        pltpu.sync_copy(aggsh.at[pl.ds(base, _RPT)],
                        out_hbm.at[cid, pl.ds(base, _RPT)])

    return k(h, src3, dst3)


def _row_mask(i):
    rows = i * _BN + lax.broadcasted_iota(jnp.int32, (_BN, 1), 0)
    return (rows < _N).astype(_f32)


def _affine_from_stats(st_ref, g_ref, b_ref):
    mu = st_ref[0:1, :] * (1.0 / _N)
    var = st_ref[1:2, :] * (1.0 / _N) - mu * mu
    cc = g_ref[...] * lax.rsqrt(var + 1e-5)
    dd = b_ref[...] - mu * cc
    return cc, dd


def _bn_apply(v, st_ref, g_ref, b_ref):
    mu = st_ref[0:1, :] * (1.0 / _N)
    var = st_ref[1:2, :] * (1.0 / _N) - mu * mu
    return (v - mu) / jnp.sqrt(var + 1e-5) * g_ref[...] + b_ref[...]


def _c1(h, a0, a1, ep, w1, b1):
    def body(ep_ref, h_ref, a0_ref, a1_ref, w1_ref, b1_ref, a_ref, st_ref, acc_ref):
        i = pl.program_id(0)
        t = ep_ref[0, 0] * h_ref[...] + a0_ref[...] + a1_ref[...]
        a = jnp.dot(t, w1_ref[...], preferred_element_type=_f32) + b1_ref[...]
        a_ref[...] = a
        am = a * _row_mask(i)

        @pl.when(i == 0)
        def _():
            acc_ref[...] = jnp.zeros((2, _D), _f32)

        acc_ref[...] += jnp.concatenate(
            [jnp.sum(am, 0, keepdims=True), jnp.sum(am * am, 0, keepdims=True)], 0)

        @pl.when(i == _NBLK - 1)
        def _():
            st_ref[...] = acc_ref[...]

    return pl.pallas_call(
        body,
        grid=(_NBLK,),
        in_specs=[
            pl.BlockSpec((1, 1), lambda i: (0, 0)),
            pl.BlockSpec((_BN, _D), lambda i: (i, 0)),
            pl.BlockSpec((_BN, _D), lambda i: (i, 0)),
            pl.BlockSpec((_BN, _D), lambda i: (i, 0)),
            pl.BlockSpec((_D, _D), lambda i: (0, 0)),
            pl.BlockSpec((1, _D), lambda i: (0, 0)),
        ],
        out_specs=[
            pl.BlockSpec((_BN, _D), lambda i: (i, 0)),
            pl.BlockSpec((2, _D), lambda i: (0, 0)),
        ],
        out_shape=[
            jax.ShapeDtypeStruct((_NP, _D), _f32),
            jax.ShapeDtypeStruct((2, _D), _f32),
        ],
        scratch_shapes=[pltpu.VMEM((2, _D), _f32)],
        compiler_params=pltpu.CompilerParams(
            dimension_semantics=("arbitrary",)),
    )(ep, h, a0, a1, w1, b1)


def _c2(a, st1, g1, be1, w2, b2):
    def body(a_ref, st_ref, g_ref, be_ref, w2_ref, b2_ref, s_ref, st2_ref, acc_ref):
        i = pl.program_id(0)
        r = jnp.maximum(_bn_apply(a_ref[...], st_ref, g_ref, be_ref), 0.0)
        s = jnp.maximum(
            jnp.dot(r, w2_ref[...], preferred_element_type=_f32) + b2_ref[...], 0.0)
        s_ref[...] = s
        sm = s * _row_mask(i)

        @pl.when(i == 0)
        def _():
            acc_ref[...] = jnp.zeros((2, _D), _f32)

        acc_ref[...] += jnp.concatenate(
            [jnp.sum(sm, 0, keepdims=True), jnp.sum(sm * sm, 0, keepdims=True)], 0)

        @pl.when(i == _NBLK - 1)
        def _():
            st2_ref[...] = acc_ref[...]

    return pl.pallas_call(
        body,
        grid=(_NBLK,),
        in_specs=[
            pl.BlockSpec((_BN, _D), lambda i: (i, 0)),
            pl.BlockSpec((2, _D), lambda i: (0, 0)),
            pl.BlockSpec((1, _D), lambda i: (0, 0)),
            pl.BlockSpec((1, _D), lambda i: (0, 0)),
            pl.BlockSpec((_D, _D), lambda i: (0, 0)),
            pl.BlockSpec((1, _D), lambda i: (0, 0)),
        ],
        out_specs=[
            pl.BlockSpec((_BN, _D), lambda i: (i, 0)),
            pl.BlockSpec((2, _D), lambda i: (0, 0)),
        ],
        out_shape=[
            jax.ShapeDtypeStruct((_NP, _D), _f32),
            jax.ShapeDtypeStruct((2, _D), _f32),
        ],
        scratch_shapes=[pltpu.VMEM((2, _D), _f32)],
        compiler_params=pltpu.CompilerParams(
            dimension_semantics=("arbitrary",)),
    )(a, st1, g1, be1, w2, b2)


def _dnorm(s, st2, go, bo):
    def body(s_ref, st_ref, g_ref, b_ref, h_ref):
        h_ref[...] = _bn_apply(s_ref[...], st_ref, g_ref, b_ref)

    return pl.pallas_call(
        body,
        grid=(_NBLK,),
        in_specs=[
            pl.BlockSpec((_BN, _D), lambda i: (i, 0)),
            pl.BlockSpec((2, _D), lambda i: (0, 0)),
            pl.BlockSpec((1, _D), lambda i: (0, 0)),
            pl.BlockSpec((1, _D), lambda i: (0, 0)),
        ],
        out_specs=pl.BlockSpec((_BN, _D), lambda i: (i, 0)),
        out_shape=jax.ShapeDtypeStruct((_NP, _D), _f32),
        compiler_params=pltpu.CompilerParams(
            dimension_semantics=("arbitrary",)),
    )(s, st2, go, bo)


def _pool(s, st2, go, bo, batch3, wh1, bh1, wh2, bh2):
    def body(s_ref, st_ref, g_ref, b_ref, bt_ref, wh1_ref, bh1_ref, wh2_ref,
             bh2_ref, out_ref, ps_ref, cnt_ref):
        i = pl.program_id(0)
        bb = bt_ref[0, 0, :]
        seg = lax.broadcasted_iota(jnp.int32, (1, _G), 1)
        oh = (bb[:, None] == seg).astype(_f32)

        @pl.when(i == 0)
        def _():
            ps_ref[...] = jnp.zeros((_G, _D), _f32)
            cnt_ref[...] = jnp.zeros((_G, 1), _f32)

        ps_ref[...] += lax.dot_general(
            oh, s_ref[...], (((0,), (0,)), ((), ())), preferred_element_type=_f32,
            precision=lax.Precision.HIGHEST)
        cnt_ref[...] += lax.dot_general(
            oh, jnp.ones((_BN, 1), _f32), (((0,), (0,)), ((), ())),
            preferred_element_type=_f32)

        @pl.when(i == _NBLK - 1)
        def _():
            cc, dd = _affine_from_stats(st_ref, g_ref, b_ref)
            pooled = ps_ref[...] * cc + cnt_ref[...] * dd
            hid = jnp.maximum(
                jnp.dot(pooled, wh1_ref[...], preferred_element_type=_f32)
                + bh1_ref[...], 0.0)
            out_ref[...] = (jnp.dot(hid, wh2_ref[...], preferred_element_type=_f32)
                            + bh2_ref[...])

    return pl.pallas_call(
        body,
        grid=(_NBLK,),
        in_specs=[
            pl.BlockSpec((_BN, _D), lambda i: (i, 0)),
            pl.BlockSpec((2, _D), lambda i: (0, 0)),
            pl.BlockSpec((1, _D), lambda i: (0, 0)),
            pl.BlockSpec((1, _D), lambda i: (0, 0)),
            pl.BlockSpec((1, 1, _BN), lambda i: (i, 0, 0)),
            pl.BlockSpec((_D, _G), lambda i: (0, 0)),
            pl.BlockSpec((1, _G), lambda i: (0, 0)),
            pl.BlockSpec((_G, 1), lambda i: (0, 0)),
            pl.BlockSpec((1, 1), lambda i: (0, 0)),
        ],
        out_specs=pl.BlockSpec((_G, 1), lambda i: (0, 0)),
        out_shape=jax.ShapeDtypeStruct((_G, 1), _f32),
        scratch_shapes=[pltpu.VMEM((_G, _D), _f32), pltpu.VMEM((_G, 1), _f32)],
        compiler_params=pltpu.CompilerParams(
            dimension_semantics=("arbitrary",)),
    )(s, st2, go, bo, batch3, wh1, bh1, wh2, bh2)


def kernel(x, edge_index, batch,
           eps_0, W1_0, b1_0, g1_0, be1_0, W2_0, b2_0, go_0, bo_0,
           eps_1, W1_1, b1_1, g1_1, be1_1, W2_1, b2_1, go_1, bo_1,
           eps_2, W1_2, b1_2, g1_2, be1_2, W2_2, b2_2, go_2, bo_2,
           Wh1, bh1, Wh2, bh2):
    src = edge_index[0].reshape(_NW, _EW)
    dst = edge_index[1].reshape(_NW, _EW)
    src3 = jnp.pad(src, ((0, 0), (0, _EWP - _EW))).reshape(_NW, _ECH, _CH)
    dst3 = jnp.pad(dst, ((0, 0), (0, _EWP - _EW)),
                   constant_values=_N).reshape(_NW, _ECH, _CH)
    xp = jnp.pad(x, ((0, _NP - _N), (0, 0)))
    batch3 = jnp.pad(batch, (0, _NP - _N),
                     constant_values=_G).reshape(_NBLK, 1, _BN)

    layers = [
        (eps_0, W1_0, b1_0, g1_0, be1_0, W2_0, b2_0, go_0, bo_0),
        (eps_1, W1_1, b1_1, g1_1, be1_1, W2_1, b2_1, go_1, bo_1),
        (eps_2, W1_2, b1_2, g1_2, be1_2, W2_2, b2_2, go_2, bo_2),
    ]
    h = xp
    out = None
    for i, (eps_i, w1, b1, g1, be1, w2, b2, go, bo) in enumerate(layers):
        agg = _sc_scatter_add(h, src3, dst3)
        ep = (1.0 + eps_i).reshape(1, 1)
        a, st1 = _c1(h, agg[0], agg[1], ep, w1, b1.reshape(1, _D))
        s, st2 = _c2(a, st1, g1.reshape(1, _D), be1.reshape(1, _D),
                     w2, b2.reshape(1, _D))
        if i < 2:
            h = _dnorm(s, st2, go.reshape(1, _D), bo.reshape(1, _D))
        else:
            out = _pool(s, st2, go.reshape(1, _D), bo.reshape(1, _D), batch3,
                        Wh1, bh1.reshape(1, _G), Wh2, bh2.reshape(1, 1))
    return jnp.squeeze(out, -1)
